# baseline (speedup 1.0000x reference)
import functools

import jax
import jax.numpy as jnp
from jax import lax
from jax.experimental import pallas as pl
from jax.experimental.pallas import tpu as pltpu
from jax.experimental.pallas import tpu_sc as plsc

NC = 2
NS = 16
NW = NC * NS
L = 16
CHUNK = 128
PACK = 8


def _mf_sc(B, H):
    assert B % (NW * CHUNK) == 0 and H == L
    bpw = B // NW
    nchunk = bpw // CHUNK

    mesh = plsc.VectorSubcoreMesh(
        core_axis_name="c", subcore_axis_name="s",
        num_cores=NC, num_subcores=NS)

    @functools.partial(
        pl.kernel,
        mesh=mesh,
        compiler_params=pltpu.CompilerParams(
            needs_layout_passes=False, use_tc_tiling_on_sc=True),
        out_type=[
            jax.ShapeDtypeStruct((B,), jnp.float32),
            jax.ShapeDtypeStruct((NW, L), jnp.float32),
        ],
        scratch_types=[
            pltpu.VMEM((bpw,), jnp.int32),
            pltpu.VMEM((bpw,), jnp.int32),
            pltpu.VMEM((nchunk, CHUNK), jnp.int32),
            pltpu.VMEM((nchunk, CHUNK), jnp.int32),
            pltpu.VMEM((nchunk, CHUNK), jnp.int32),
            pltpu.VMEM((nchunk, CHUNK), jnp.int32),
            pltpu.VMEM((2, CHUNK, 128), jnp.float32),
            pltpu.VMEM((2, CHUNK, 128), jnp.float32),
            pltpu.VMEM((nchunk, CHUNK), jnp.float32),
            pltpu.VMEM((nchunk, CHUNK), jnp.float32),
            pltpu.VMEM((bpw,), jnp.float32),
            pltpu.VMEM((bpw,), jnp.float32),
            pltpu.VMEM((L,), jnp.float32),
            pltpu.VMEM((1,), jnp.float32),
            pltpu.SemaphoreType.DMA,
            pltpu.SemaphoreType.DMA,
            pltpu.SemaphoreType.DMA,
            pltpu.SemaphoreType.DMA,
            pltpu.SemaphoreType.DMA,
        ],
    )
    def k(user_h, item_h, rating_h, uw_h, iw_h, ub_h, ib_h, bias_h,
          out_h, part_h,
          idx1_u, idx1_i, idxd_u, idxd_i, idxr_u, idxr_i,
          rows_u, rows_i, bu, bi, rat_v, out_v, loss_v, bias_v,
          sem0, *csem):
        wid = lax.axis_index("s") * NC + lax.axis_index("c")
        base = wid * bpw
        iota = lax.iota(jnp.int32, L)

        cps = [
            pltpu.async_copy(user_h.at[pl.ds(base, bpw)], idx1_u, sem0),
            pltpu.async_copy(item_h.at[pl.ds(base, bpw)], idx1_i, sem0),
            pltpu.async_copy(rating_h.at[pl.ds(base, bpw)], rat_v, sem0),
            pltpu.async_copy(bias_h, bias_v, sem0),
        ]
        for cp in cps:
            cp.wait()

        for c in range(nchunk):
            def mk_idx(g, _, c=c):
                s = pl.multiple_of(g * L, L)
                ru = idx1_u[pl.ds(c * CHUNK + s, L)]
                ri = idx1_i[pl.ds(c * CHUNK + s, L)]
                idxr_u[c, pl.ds(s, L)] = ru
                idxr_i[c, pl.ds(s, L)] = ri
                idxd_u[c, pl.ds(s, L)] = lax.shift_right_logical(ru, 3)
                idxd_i[c, pl.ds(s, L)] = lax.shift_right_logical(ri, 3)
                return 0
            lax.fori_loop(0, CHUNK // L, mk_idx, 0)

        def fire(c):
            sem = csem[c]
            b = c % 2
            return [
                pltpu.async_copy(uw_h.at[idxd_u.at[c]], rows_u.at[b], sem),
                pltpu.async_copy(iw_h.at[idxd_i.at[c]], rows_i.at[b], sem),
                pltpu.async_copy(ub_h.at[idxr_u.at[c]], bu.at[c], sem),
                pltpu.async_copy(ib_h.at[idxr_i.at[c]], bi.at[c], sem),
            ]

        inflight = {0: fire(0)}
        if nchunk > 1:
            inflight[1] = fire(1)

        bias_bc = plsc.load_gather(bias_v, [jnp.zeros((L,), jnp.int32)])
        loss_vec = jnp.zeros((L,), jnp.float32)
        for c in range(nchunk):
            for cp in inflight.pop(c):
                cp.wait()
            buf = jnp.full((L,), c % 2, jnp.int32)

            def group(g, lv, c=c, buf=buf):
                s = pl.multiple_of(g * L, L)
                bloc = iota + s
                ru = idx1_u[pl.ds(c * CHUNK + s, L)]
                ri = idx1_i[pl.ds(c * CHUNK + s, L)]
                cu = lax.shift_left(jnp.bitwise_and(ru, 7), 4)
                ci = lax.shift_left(jnp.bitwise_and(ri, 7), 4)
                ubv = bu[c, pl.ds(s, L)]
                ibv = bi[c, pl.ds(s, L)]
                acc = jnp.zeros((L,), jnp.float32)
                for h in range(H):
                    uv = plsc.load_gather(rows_u, [buf, bloc, cu + h]) + ubv
                    iv = plsc.load_gather(rows_i, [buf, bloc, ci + h]) + ibv
                    acc = acc + uv * iv
                tgt = acc + bias_bc
                out_v[pl.ds(c * CHUNK + s, L)] = tgt
                d = tgt - rat_v[pl.ds(c * CHUNK + s, L)]
                return lv + d * d

            loss_vec = lax.fori_loop(0, CHUNK // L, group, loss_vec)
            if c + 2 < nchunk:
                inflight[c + 2] = fire(c + 2)

        loss_v[...] = loss_vec
        pltpu.sync_copy(out_v, out_h.at[pl.ds(base, bpw)])
        pltpu.sync_copy(loss_v, part_h.at[wid])

    return k


def kernel(user, item, rating, user_weight, item_weight, user_bias,
           item_bias, bias):
    B = user.shape[0]
    H = user_weight.shape[1]
    uwr, iwr = lax.optimization_barrier(
        (user_weight.reshape(-1, PACK * H),
         item_weight.reshape(-1, PACK * H)))
    target, partials = _mf_sc(B, H)(
        user, item, rating, uwr, iwr,
        user_bias.reshape(-1), item_bias.reshape(-1), bias)
    loss = jnp.sum(partials) / B
    return target, loss
